# baseline (speedup 1.0000x reference)
import functools

import jax
import jax.numpy as jnp
from jax import lax
from jax.experimental import pallas as pl
from jax.experimental.pallas import tpu as pltpu
from jax.experimental.pallas import tpu_sc as plsc

F = 26
V1 = 100001
D = 16
B = 16384
DD = 13
FD = F * D
NJ = 4

NC, NS = 2, 16
NW = NC * NS
RW = B // NW
CH = 64
NCHUNK = RW // CH
NSLOT = 3


def _sc_gather(idx_t, emb, lin2):
    mesh = plsc.VectorSubcoreMesh(
        core_axis_name="c", subcore_axis_name="s", num_cores=NC, num_subcores=NS)

    @functools.partial(
        pl.kernel,
        out_type=(
            jax.ShapeDtypeStruct((B, NJ, 128), jnp.float32),
            jax.ShapeDtypeStruct((B,), jnp.float32),
        ),
        mesh=mesh,
        scratch_types=[
            pltpu.VMEM((F, RW), jnp.int32),
            pltpu.VMEM((NSLOT, F, CH, D), jnp.float32),
            pltpu.VMEM((F, RW), jnp.float32),
            pltpu.VMEM((RW,), jnp.float32),
            pltpu.SemaphoreType.DMA,
            pltpu.SemaphoreType.DMA,
            pltpu.SemaphoreType.DMA,
        ],
        compiler_params=pltpu.CompilerParams(use_tc_tiling_on_sc=False),
    )
    def k(idx_hbm, emb_hbm, lin_hbm, e_out, ls_out, idx_v, gbuf, lbuf,
          lsum_v, sem_e, sem_l, sem_o):
        wid = lax.axis_index("s") * NC + lax.axis_index("c")
        row0 = wid * RW

        def fire_g(c, _):
            s = lax.rem(c, NSLOT)

            def f_body(f, carry):
                pltpu.make_async_copy(
                    emb_hbm.at[f].at[idx_v.at[f, pl.ds(c * CH, CH)]],
                    gbuf.at[s, f], sem_e).start()
                return carry

            lax.fori_loop(0, F, f_body, 0)
            return _

        def drain_g(c, _):
            s = lax.rem(c, NSLOT)

            def f_body(f, carry):
                pltpu.make_async_copy(
                    emb_hbm.at[f].at[idx_v.at[f, pl.ds(c * CH, CH)]],
                    gbuf.at[s, f], sem_e).wait()
                return carry

            lax.fori_loop(0, F, f_body, 0)
            return _

        def fire_out(c, _):
            s = lax.rem(c, NSLOT)

            def f_body(f, carry):
                j = f // D
                pltpu.make_async_copy(
                    gbuf.at[s, f],
                    e_out.at[pl.ds(row0 + c * CH, CH), lax.div(f, 8),
                             pl.ds(lax.rem(f, 8) * D, D)],
                    sem_o).start()
                return carry

            lax.fori_loop(0, F, f_body, 0)
            return _

        def drain_out(c, _):
            s = lax.rem(c, NSLOT)

            def f_body(f, carry):
                pltpu.make_async_copy(
                    gbuf.at[s, f],
                    e_out.at[pl.ds(row0 + c * CH, CH), lax.div(f, 8),
                             pl.ds(lax.rem(f, 8) * D, D)],
                    sem_o).wait()
                return carry

            lax.fori_loop(0, F, f_body, 0)
            return _

        pltpu.sync_copy(idx_hbm.at[:, pl.ds(row0, RW)], idx_v)

        def fire_lin(f, carry):
            pltpu.make_async_copy(
                lin_hbm.at[f].at[idx_v.at[f]], lbuf.at[f], sem_l).start()
            return carry

        lax.fori_loop(0, F, fire_lin, 0)

        fire_g(0, 0)
        fire_g(1, 0)

        def cgroup(c, carry):
            drain_g(c, 0)

            def _do():
                drain_out(c - 1, 0)

            def _fg():
                fire_g(c + 2, 0)

            pl.when(c >= 1)(_do)
            fire_out(c, 0)
            pl.when(c + 2 < NCHUNK)(_fg)
            return carry

        lax.fori_loop(0, NCHUNK, cgroup, 0)
        drain_out(NCHUNK - 1, 0)

        def drain_lin(f, carry):
            pltpu.make_async_copy(
                lin_hbm.at[f].at[idx_v.at[f]], lbuf.at[f], sem_l).wait()
            return carry

        lax.fori_loop(0, F, drain_lin, 0)

        for j in range(RW // 16):
            sl = pl.ds(j * 16, 16)

            def acc_f(f, acc):
                return acc + lbuf[f, sl]

            lsum_v[sl] = lax.fori_loop(
                0, F, acc_f, jnp.zeros((16,), jnp.float32))

        pltpu.sync_copy(lsum_v, ls_out.at[pl.ds(row0, RW)])

    return k(idx_t, emb, lin2)


RB = 1024


def _mlp_body(e_ref, ls_ref, d_ref, a1e_ref, a1d_ref, c1_ref, a2_ref, c2_ref,
              w3_ref, ldw_ref, cadd_ref, o_ref):
    lane = lax.broadcasted_iota(jnp.int32, (RB, 128), 1)
    mask3 = (lane < 32).astype(jnp.float32)
    ki = lax.broadcasted_iota(jnp.int32, (128, D), 0) % D
    di = lax.broadcasted_iota(jnp.int32, (128, D), 1)
    msum = (ki == di).astype(jnp.float32)
    dd = d_ref[...]
    h1 = jnp.dot(dd, a1d_ref[...]) + c1_ref[...]
    s = jnp.zeros((RB, D), jnp.float32)
    sq = jnp.zeros((RB,), jnp.float32)
    for j in range(NJ):
        e_j = e_ref[:, j, :]
        if j == NJ - 1:
            e_j = e_j * mask3
        h1 = h1 + jnp.dot(e_j, a1e_ref[pl.ds(j * 128, 128), :])
        s = s + jnp.dot(e_j, msum)
        sq = sq + jnp.sum(e_j * e_j, axis=1)
    h1 = jnp.maximum(h1, 0.0)
    h2 = jnp.maximum(jnp.dot(h1, a2_ref[...]) + c2_ref[...], 0.0)
    deep = jnp.sum(h2 * w3_ref[...], axis=1)
    fm = 0.5 * (jnp.sum(s * s, axis=1) - sq)
    ld = jnp.sum(dd * ldw_ref[...], axis=1)
    o_ref[...] = ls_ref[...] + ld + fm + deep + cadd_ref[0]


def _tc_mlp(e, ls, dense, a1e, a1d, c1, a2, c2, w3, ldw, cadd):
    grid = (B // RB,)
    return pl.pallas_call(
        _mlp_body,
        grid=grid,
        in_specs=[
            pl.BlockSpec((RB, NJ, 128), lambda i: (i, 0, 0)),
            pl.BlockSpec((RB,), lambda i: (i,)),
            pl.BlockSpec((RB, DD), lambda i: (i, 0)),
            pl.BlockSpec((512, 128), lambda i: (0, 0)),
            pl.BlockSpec((DD, 128), lambda i: (0, 0)),
            pl.BlockSpec((1, 128), lambda i: (0, 0)),
            pl.BlockSpec((128, 64), lambda i: (0, 0)),
            pl.BlockSpec((1, 64), lambda i: (0, 0)),
            pl.BlockSpec((1, 64), lambda i: (0, 0)),
            pl.BlockSpec((1, DD), lambda i: (0, 0)),
            pl.BlockSpec(memory_space=pltpu.SMEM),
        ],
        out_specs=pl.BlockSpec((RB,), lambda i: (i,)),
        out_shape=jax.ShapeDtypeStruct((B,), jnp.float32),
        compiler_params=pltpu.CompilerParams(
            dimension_semantics=("parallel",)),
    )(e, ls, dense, a1e, a1d, c1, a2, c2, w3, ldw, cadd)


def kernel(sparse_inputs, dense_inputs, emb_tables, lin_tables, ld_W, ld_b,
           bn0_g, bn0_b, W1, b1, bn1_g, bn1_b, W2, b2, bn2_g, bn2_b,
           Wout, bout, bias):
    idx_t = sparse_inputs.astype(jnp.int32).T
    lin2 = lin_tables[:, :, 0]

    e, ls = _sc_gather(idx_t, emb_tables, lin2)

    s0 = 1.0 / jnp.sqrt(1.0 + 1e-5)
    g0 = bn0_g * s0
    w1f = W1 * g0[None, :]
    b1f = b1 + W1 @ (bn0_b * s0)
    s1 = bn1_g * s0
    w1ff = w1f * s1[:, None]
    c1 = (b1f * s1 + bn1_b)[None, :]
    s2 = bn2_g * s0
    a2 = (W2 * s2[:, None]).T
    c2 = (b2 * s2 + bn2_b)[None, :]
    a1e = jnp.zeros((512, 128), jnp.float32).at[:FD].set(w1ff[:, :FD].T)
    a1d = w1ff[:, FD:].T
    w3 = Wout
    ldw = ld_W
    cadd = (bias + ld_b + bout).reshape(1)

    return _tc_mlp(e, ls, dense_inputs, a1e, a1d, c1, a2, c2, w3, ldw, cadd)


# speedup vs baseline: 1.4328x; 1.4328x over previous
import functools

import jax
import jax.numpy as jnp
from jax import lax
from jax.experimental import pallas as pl
from jax.experimental.pallas import tpu as pltpu
from jax.experimental.pallas import tpu_sc as plsc

F = 26
V1 = 100001
D = 16
B = 16384
DD = 13
FD = F * D

NC, NS = 2, 16
NW = NC * NS
CW = B // NW
NG = CW // 16
NSLOT = 3


def _sc_gather(idx_t, emb_t, lin_t):
    mesh = plsc.VectorSubcoreMesh(
        core_axis_name="c", subcore_axis_name="s", num_cores=NC, num_subcores=NS)

    @functools.partial(
        pl.kernel,
        out_type=(
            jax.ShapeDtypeStruct((FD, B), jnp.float32),
            jax.ShapeDtypeStruct((B,), jnp.float32),
        ),
        mesh=mesh,
        scratch_types=[
            pltpu.VMEM((F, CW), jnp.int32),
            pltpu.VMEM((NSLOT, F, CW), jnp.float32),
            pltpu.VMEM((F, CW), jnp.float32),
            pltpu.VMEM((CW,), jnp.float32),
            pltpu.SemaphoreType.DMA((NSLOT,)),
            pltpu.SemaphoreType.DMA,
            pltpu.SemaphoreType.DMA,
        ],
        compiler_params=pltpu.CompilerParams(use_tc_tiling_on_sc=False),
    )
    def k(idx_hbm, emb_hbm, lin_hbm, et_out, ls_out, idx_v, gbuf, lbuf,
          lsum_v, sem_e, sem_l, sem_o):
        wid = lax.axis_index("s") * NC + lax.axis_index("c")
        col = wid * CW

        def fire_g(d, _):
            s = lax.rem(d, NSLOT)

            def f_body(f, carry):
                def g_body(g, carry2):
                    iv = idx_v[f, pl.ds(g * 16, 16)]
                    pltpu.make_async_copy(
                        emb_hbm.at[f, d].at[iv],
                        gbuf.at[s, f, pl.ds(g * 16, 16)],
                        sem_e.at[s]).start()
                    return carry2

                lax.fori_loop(0, NG, g_body, 0)
                return carry

            lax.fori_loop(0, F, f_body, 0)
            return _

        def drain_g(d, _):
            s = lax.rem(d, NSLOT)

            def f_body(f, carry):
                pltpu.make_async_copy(
                    et_out.at[0, pl.ds(0, CW)], gbuf.at[s, f],
                    sem_e.at[s]).wait()
                return carry

            lax.fori_loop(0, F, f_body, 0)
            return _

        def fire_out(d, _):
            s = lax.rem(d, NSLOT)

            def f_body(f, carry):
                pltpu.make_async_copy(
                    gbuf.at[s, f], et_out.at[f * D + d, pl.ds(col, CW)],
                    sem_o).start()
                return carry

            lax.fori_loop(0, F, f_body, 0)
            return _

        def drain_out(d, _):
            s = lax.rem(d, NSLOT)

            def f_body(f, carry):
                pltpu.make_async_copy(
                    gbuf.at[s, f], et_out.at[f * D + d, pl.ds(col, CW)],
                    sem_o).wait()
                return carry

            lax.fori_loop(0, F, f_body, 0)
            return _

        pltpu.sync_copy(idx_hbm.at[:, pl.ds(col, CW)], idx_v)

        def fire_lin(f, carry):
            def g_body(g, carry2):
                iv = idx_v[f, pl.ds(g * 16, 16)]
                pltpu.make_async_copy(
                    lin_hbm.at[f, 0].at[iv],
                    lbuf.at[f, pl.ds(g * 16, 16)], sem_l).start()
                return carry2

            lax.fori_loop(0, NG, g_body, 0)
            return carry

        lax.fori_loop(0, F, fire_lin, 0)

        fire_g(0, 0)
        fire_g(1, 0)

        def dgroup(d, carry):
            drain_g(d, 0)

            def _do():
                drain_out(d - 1, 0)

            def _fg():
                fire_g(d + 2, 0)

            pl.when(d >= 1)(_do)
            fire_out(d, 0)
            pl.when(d + 2 < D)(_fg)
            return carry

        lax.fori_loop(0, D, dgroup, 0)
        drain_out(D - 1, 0)

        def drain_lin(f, carry):
            pltpu.make_async_copy(
                et_out.at[0, pl.ds(0, CW)], lbuf.at[f], sem_l).wait()
            return carry

        lax.fori_loop(0, F, drain_lin, 0)

        for j in range(CW // 16):
            sl = pl.ds(j * 16, 16)

            def acc_f(f, acc):
                return acc + lbuf[f, sl]

            lsum_v[sl] = lax.fori_loop(
                0, F, acc_f, jnp.zeros((16,), jnp.float32))

        pltpu.sync_copy(lsum_v, ls_out.at[pl.ds(col, CW)])

    return k(idx_t, emb_t, lin_t)


RB = 1024
NP = RB // 128


def _mlp_body(e_ref, ls_ref, d_ref, a1e_ref, a1d_ref, c1_ref, a2_ref, c2_ref,
              w3_ref, ldw_ref, cadd_ref, o_ref):
    ri = lax.broadcasted_iota(jnp.int32, (D, FD), 0)
    ki = lax.broadcasted_iota(jnp.int32, (D, FD), 1) % D
    fsum = (ri == ki).astype(jnp.float32)
    for p in range(NP):
        sl = pl.ds(p * 128, 128)
        e = e_ref[:, p, :]
        dd = d_ref[:, sl]
        h1 = jnp.maximum(
            jnp.dot(a1e_ref[...], e) + jnp.dot(a1d_ref[...], dd)
            + c1_ref[...], 0.0)
        h2 = jnp.maximum(jnp.dot(a2_ref[...], h1) + c2_ref[...], 0.0)
        deep = jnp.sum(h2 * w3_ref[...], axis=0)
        s = jnp.dot(fsum, e)
        fm = 0.5 * (jnp.sum(s * s, axis=0) - jnp.sum(e * e, axis=0))
        ld = jnp.sum(dd * ldw_ref[...], axis=0)
        o_ref[sl] = ls_ref[sl] + ld + fm + deep + cadd_ref[0]


def _tc_mlp(e3, ls, dense_t, a1e, a1d, c1, a2, c2, w3, ldw, cadd):
    grid = (B // RB,)
    return pl.pallas_call(
        _mlp_body,
        grid=grid,
        in_specs=[
            pl.BlockSpec((FD, NP, 128), lambda i: (0, i, 0)),
            pl.BlockSpec((RB,), lambda i: (i,)),
            pl.BlockSpec((DD, RB), lambda i: (0, i)),
            pl.BlockSpec((128, FD), lambda i: (0, 0)),
            pl.BlockSpec((128, DD), lambda i: (0, 0)),
            pl.BlockSpec((128, 1), lambda i: (0, 0)),
            pl.BlockSpec((64, 128), lambda i: (0, 0)),
            pl.BlockSpec((64, 1), lambda i: (0, 0)),
            pl.BlockSpec((64, 1), lambda i: (0, 0)),
            pl.BlockSpec((DD, 1), lambda i: (0, 0)),
            pl.BlockSpec(memory_space=pltpu.SMEM),
        ],
        out_specs=pl.BlockSpec((RB,), lambda i: (i,)),
        out_shape=jax.ShapeDtypeStruct((B,), jnp.float32),
        compiler_params=pltpu.CompilerParams(
            dimension_semantics=("parallel",)),
    )(e3, ls, dense_t, a1e, a1d, c1, a2, c2, w3, ldw, cadd)


def kernel(sparse_inputs, dense_inputs, emb_tables, lin_tables, ld_W, ld_b,
           bn0_g, bn0_b, W1, b1, bn1_g, bn1_b, W2, b2, bn2_g, bn2_b,
           Wout, bout, bias):
    idx_t = sparse_inputs.astype(jnp.int32).T
    emb_t = jnp.transpose(emb_tables, (0, 2, 1))
    lin_t = jnp.transpose(lin_tables, (0, 2, 1))
    dense_t = dense_inputs.T

    e_t, ls = _sc_gather(idx_t, emb_t, lin_t)
    e3 = e_t.reshape(FD, B // 128, 128)

    s0 = 1.0 / jnp.sqrt(1.0 + 1e-5)
    g0 = bn0_g * s0
    w1f = W1 * g0[None, :]
    b1f = b1 + W1 @ (bn0_b * s0)
    s1 = bn1_g * s0
    w1ff = w1f * s1[:, None]
    c1 = (b1f * s1 + bn1_b)[:, None]
    s2 = bn2_g * s0
    a2 = W2 * s2[:, None]
    c2 = (b2 * s2 + bn2_b)[:, None]
    a1e = w1ff[:, :FD]
    a1d = w1ff[:, FD:]
    w3 = Wout.reshape(64, 1)
    ldw = ld_W.reshape(DD, 1)
    cadd = (bias + ld_b + bout).reshape(1)

    return _tc_mlp(e3, ls, dense_t, a1e, a1d, c1, a2, c2, w3, ldw, cadd)


# speedup vs baseline: 6.6593x; 4.6478x over previous
import functools

import jax
import jax.numpy as jnp
from jax import lax
from jax.experimental import pallas as pl
from jax.experimental.pallas import tpu as pltpu
from jax.experimental.pallas import tpu_sc as plsc

F = 26
V1 = 100001
D = 16
B = 16384
DD = 13
FD = F * D

NC, NS = 2, 16
NW = NC * NS
CW = B // NW
NG = CW // 16
NSLOT = 3

VT = 782
NTILE = F * 2 * VT
TPW = (NTILE + NW - 1) // NW
NRING = 12


def _sc_detile(emb, tailp):
    mesh = plsc.VectorSubcoreMesh(
        core_axis_name="c", subcore_axis_name="s", num_cores=NC, num_subcores=NS)

    @functools.partial(
        pl.kernel,
        out_type=jax.ShapeDtypeStruct((NTILE, 8, 128), jnp.float32),
        mesh=mesh,
        scratch_types=[
            pltpu.VMEM((NRING, 8, 128), jnp.float32),
            pltpu.SemaphoreType.DMA,
            pltpu.SemaphoreType.DMA,
        ],
    )
    def k(emb_hbm, tailp_hbm, t_out, vbuf, sem_r, sem_w):
        wid = lax.axis_index("s") * NC + lax.axis_index("c")
        LAG = 6

        def fgt(kk):
            tid = wid * TPW + kk
            f = lax.div(tid, 2 * VT)
            r = lax.rem(tid, 2 * VT)
            return tid, f, lax.div(r, VT), lax.rem(r, VT)

        def read_desc_full(kk):
            tid, f, t, g = fgt(kk)
            return pltpu.make_async_copy(
                emb_hbm.at[f, pl.ds(pl.multiple_of(t * 8, 8), 8),
                           pl.ds(pl.multiple_of(g * 128, 128), 128)],
                vbuf.at[lax.rem(kk, NRING)], sem_r)

        def read_desc_tail(kk):
            tid, f, t, g = fgt(kk)
            return pltpu.make_async_copy(
                tailp_hbm.at[f, pl.ds(pl.multiple_of(t * 8, 8), 8), :],
                vbuf.at[lax.rem(kk, NRING)], sem_r)

        def start_read(kk):
            tid, f, t, g = fgt(kk)
            pl.when(jnp.logical_and(tid < NTILE, g < VT - 1))(
                lambda: read_desc_full(kk).start())
            pl.when(jnp.logical_and(tid < NTILE, g == VT - 1))(
                lambda: read_desc_tail(kk).start())

        def finish_read_start_write(kk):
            tid, f, t, g = fgt(kk)

            def _full():
                read_desc_full(kk).wait()
                pltpu.make_async_copy(
                    vbuf.at[lax.rem(kk, NRING)], t_out.at[tid], sem_w).start()

            def _tail():
                read_desc_tail(kk).wait()
                pltpu.make_async_copy(
                    vbuf.at[lax.rem(kk, NRING)], t_out.at[tid], sem_w).start()

            pl.when(jnp.logical_and(tid < NTILE, g < VT - 1))(_full)
            pl.when(jnp.logical_and(tid < NTILE, g == VT - 1))(_tail)

        def drain_w(kk):
            def _w():
                pltpu.make_async_copy(
                    vbuf.at[lax.rem(kk, NRING)],
                    t_out.at[wid * TPW + kk], sem_w).wait()

            pl.when(wid * TPW + kk < NTILE)(_w)

        def body(kk, carry):
            pl.when(kk >= NRING)(lambda: drain_w(kk - NRING))
            start_read(kk)
            pl.when(kk >= LAG)(lambda: finish_read_start_write(kk - LAG))
            return carry

        lax.fori_loop(0, TPW, body, 0)

        def tail1(kk, carry):
            finish_read_start_write(kk)
            return carry

        lax.fori_loop(TPW - LAG, TPW, tail1, 0)

        def tail2(kk, carry):
            drain_w(kk)
            return carry

        lax.fori_loop(TPW - NRING, TPW, tail2, 0)

    return k(emb, tailp)


def _sc_gather(idx_t, e1d, lin_t):
    mesh = plsc.VectorSubcoreMesh(
        core_axis_name="c", subcore_axis_name="s", num_cores=NC, num_subcores=NS)

    @functools.partial(
        pl.kernel,
        out_type=(
            jax.ShapeDtypeStruct((FD, B), jnp.float32),
            jax.ShapeDtypeStruct((B,), jnp.float32),
        ),
        mesh=mesh,
        scratch_types=[
            pltpu.VMEM((F, CW), jnp.int32),
            pltpu.VMEM((NSLOT, F, CW), jnp.float32),
            pltpu.VMEM((F, CW), jnp.float32),
            pltpu.VMEM((CW,), jnp.float32),
            pltpu.SemaphoreType.DMA((NSLOT,)),
            pltpu.SemaphoreType.DMA,
            pltpu.SemaphoreType.DMA,
        ],
        compiler_params=pltpu.CompilerParams(use_tc_tiling_on_sc=False),
    )
    def k(idx_hbm, emb_hbm, lin_hbm, et_out, ls_out, idx_v, gbuf, lbuf,
          lsum_v, sem_e, sem_l, sem_o):
        wid = lax.axis_index("s") * NC + lax.axis_index("c")
        col = wid * CW

        def fire_g(d, _):
            s = lax.rem(d, NSLOT)
            base = lax.div(d, 8) * (VT * 1024) + lax.rem(d, 8) * 128

            def f_body(f, carry):
                fbase = base + f * (2 * VT * 1024)

                def g_body(g, carry2):
                    iv = idx_v[f, pl.ds(g * 16, 16)]
                    w = fbase + iv + (iv >> 7) * 896
                    pltpu.make_async_copy(
                        emb_hbm.at[w],
                        gbuf.at[s, f, pl.ds(g * 16, 16)],
                        sem_e.at[s]).start()
                    return carry2

                lax.fori_loop(0, NG, g_body, 0)
                return carry

            lax.fori_loop(0, F, f_body, 0)
            return _

        def drain_g(d, _):
            s = lax.rem(d, NSLOT)

            def f_body(f, carry):
                pltpu.make_async_copy(
                    et_out.at[0, pl.ds(0, CW)], gbuf.at[s, f],
                    sem_e.at[s]).wait()
                return carry

            lax.fori_loop(0, F, f_body, 0)
            return _

        def fire_out(d, _):
            s = lax.rem(d, NSLOT)

            def f_body(f, carry):
                pltpu.make_async_copy(
                    gbuf.at[s, f], et_out.at[f * D + d, pl.ds(col, CW)],
                    sem_o).start()
                return carry

            lax.fori_loop(0, F, f_body, 0)
            return _

        def drain_out(d, _):
            s = lax.rem(d, NSLOT)

            def f_body(f, carry):
                pltpu.make_async_copy(
                    gbuf.at[s, f], et_out.at[f * D + d, pl.ds(col, CW)],
                    sem_o).wait()
                return carry

            lax.fori_loop(0, F, f_body, 0)
            return _

        pltpu.sync_copy(idx_hbm.at[:, pl.ds(col, CW)], idx_v)

        def fire_lin(f, carry):
            def g_body(g, carry2):
                iv = idx_v[f, pl.ds(g * 16, 16)]
                pltpu.make_async_copy(
                    lin_hbm.at[f, 0].at[iv],
                    lbuf.at[f, pl.ds(g * 16, 16)], sem_l).start()
                return carry2

            lax.fori_loop(0, NG, g_body, 0)
            return carry

        lax.fori_loop(0, F, fire_lin, 0)

        fire_g(0, 0)
        fire_g(1, 0)

        def dgroup(d, carry):
            drain_g(d, 0)

            def _do():
                drain_out(d - 1, 0)

            def _fg():
                fire_g(d + 2, 0)

            pl.when(d >= 1)(_do)
            fire_out(d, 0)
            pl.when(d + 2 < D)(_fg)
            return carry

        lax.fori_loop(0, D, dgroup, 0)
        drain_out(D - 1, 0)

        def drain_lin(f, carry):
            pltpu.make_async_copy(
                et_out.at[0, pl.ds(0, CW)], lbuf.at[f], sem_l).wait()
            return carry

        lax.fori_loop(0, F, drain_lin, 0)

        for j in range(CW // 16):
            sl = pl.ds(j * 16, 16)

            def acc_f(f, acc):
                return acc + lbuf[f, sl]

            lsum_v[sl] = lax.fori_loop(
                0, F, acc_f, jnp.zeros((16,), jnp.float32))

        pltpu.sync_copy(lsum_v, ls_out.at[pl.ds(col, CW)])

    return k(idx_t, e1d, lin_t)


RB = 1024
NP = RB // 128


def _mlp_body(e_ref, ls_ref, d_ref, a1e_ref, a1d_ref, c1_ref, a2_ref, c2_ref,
              w3_ref, ldw_ref, cadd_ref, o_ref):
    ri = lax.broadcasted_iota(jnp.int32, (D, FD), 0)
    ki = lax.broadcasted_iota(jnp.int32, (D, FD), 1) % D
    fsum = (ri == ki).astype(jnp.float32)
    for p in range(NP):
        sl = pl.ds(p * 128, 128)
        e = e_ref[:, p, :]
        dd = d_ref[:, sl]
        h1 = jnp.maximum(
            jnp.dot(a1e_ref[...], e) + jnp.dot(a1d_ref[...], dd)
            + c1_ref[...], 0.0)
        h2 = jnp.maximum(jnp.dot(a2_ref[...], h1) + c2_ref[...], 0.0)
        deep = jnp.sum(h2 * w3_ref[...], axis=0)
        s = jnp.dot(fsum, e)
        fm = 0.5 * (jnp.sum(s * s, axis=0) - jnp.sum(e * e, axis=0))
        ld = jnp.sum(dd * ldw_ref[...], axis=0)
        o_ref[sl] = ls_ref[sl] + ld + fm + deep + cadd_ref[0]


def _tc_mlp(e3, ls, dense_t, a1e, a1d, c1, a2, c2, w3, ldw, cadd):
    grid = (B // RB,)
    return pl.pallas_call(
        _mlp_body,
        grid=grid,
        in_specs=[
            pl.BlockSpec((FD, NP, 128), lambda i: (0, i, 0)),
            pl.BlockSpec((RB,), lambda i: (i,)),
            pl.BlockSpec((DD, RB), lambda i: (0, i)),
            pl.BlockSpec((128, FD), lambda i: (0, 0)),
            pl.BlockSpec((128, DD), lambda i: (0, 0)),
            pl.BlockSpec((128, 1), lambda i: (0, 0)),
            pl.BlockSpec((64, 128), lambda i: (0, 0)),
            pl.BlockSpec((64, 1), lambda i: (0, 0)),
            pl.BlockSpec((64, 1), lambda i: (0, 0)),
            pl.BlockSpec((DD, 1), lambda i: (0, 0)),
            pl.BlockSpec(memory_space=pltpu.SMEM),
        ],
        out_specs=pl.BlockSpec((RB,), lambda i: (i,)),
        out_shape=jax.ShapeDtypeStruct((B,), jnp.float32),
        compiler_params=pltpu.CompilerParams(
            dimension_semantics=("parallel",)),
    )(e3, ls, dense_t, a1e, a1d, c1, a2, c2, w3, ldw, cadd)


def kernel(sparse_inputs, dense_inputs, emb_tables, lin_tables, ld_W, ld_b,
           bn0_g, bn0_b, W1, b1, bn1_g, bn1_b, W2, b2, bn2_g, bn2_b,
           Wout, bout, bias):
    idx_t = sparse_inputs.astype(jnp.int32).T
    emb_t = jnp.transpose(emb_tables, (0, 2, 1))
    lin_t = jnp.transpose(lin_tables, (0, 2, 1))
    dense_t = dense_inputs.T

    tailp = jnp.pad(emb_t[:, :, (VT - 1) * 128:],
                    ((0, 0), (0, 0), (0, VT * 128 - V1)))
    e1d = _sc_detile(emb_t, tailp).reshape(NTILE * 1024)
    e_t, ls = _sc_gather(idx_t, e1d, lin_t)
    e3 = e_t.reshape(FD, B // 128, 128)

    s0 = 1.0 / jnp.sqrt(1.0 + 1e-5)
    g0 = bn0_g * s0
    w1f = W1 * g0[None, :]
    b1f = b1 + W1 @ (bn0_b * s0)
    s1 = bn1_g * s0
    w1ff = w1f * s1[:, None]
    c1 = (b1f * s1 + bn1_b)[:, None]
    s2 = bn2_g * s0
    a2 = W2 * s2[:, None]
    c2 = (b2 * s2 + bn2_b)[:, None]
    a1e = w1ff[:, :FD]
    a1d = w1ff[:, FD:]
    w3 = Wout.reshape(64, 1)
    ldw = ld_W.reshape(DD, 1)
    cadd = (bias + ld_b + bout).reshape(1)

    return _tc_mlp(e3, ls, dense_t, a1e, a1d, c1, a2, c2, w3, ldw, cadd)
